# baseline (speedup 1.0000x reference)
import jax
import jax.numpy as jnp
from jax import lax
from jax.experimental import pallas as pl
from jax.experimental.pallas import tpu as pltpu
from jax.experimental.pallas import tpu_sc as plsc

_TABLE_ROWS = 256 * 256 * 256
_S = 512 * 512
_P = 4 * _S
_NW = 32
_PW = _P // _NW
_C = 8192
_NCHUNK = _PW // _C
_L = 16


def _body(img_hbm, w_hbm, b_hbm, out_hbm,
          xr, xg, xb, i0, i1, i2, w0, w1, w2, b0, b1, b2, sem):
    nc = 2
    wid = lax.axis_index("s") * nc + lax.axis_index("c")
    batch = wid // 8
    base = (wid % 8) * _PW
    x3 = (xr, xg, xb)
    idx3 = (i0, i1, i2)
    wv3 = (w0, w1, w2)
    bv3 = (b0, b1, b2)

    for k in range(_NCHUNK):
        off = base + k * _C

        for ch in range(3):
            pltpu.sync_copy(img_hbm.at[3 * batch + ch, pl.ds(off, _C)],
                            x3[ch])

        def idx_body(i, _):
            sl = pl.ds(i * _L, _L)
            rr = (xr[sl] + 1.0) * 127.5
            gg = (xg[sl] + 1.0) * 127.5
            bb = (xb[sl] + 1.0) * 127.5
            t = rr * 65536.0 + gg * 256.0 + bb
            iv = t.astype(jnp.int32)
            iv = jnp.clip(iv, 0, _TABLE_ROWS - 1)
            iv3 = iv * 3
            i0[sl] = iv3
            i1[sl] = iv3 + 1
            i2[sl] = iv3 + 2
            return 0

        lax.fori_loop(0, _C // _L, idx_body, 0, unroll=4)

        for ch in range(3):
            pltpu.async_copy(w_hbm.at[idx3[ch]], wv3[ch], sem)
            pltpu.async_copy(b_hbm.at[idx3[ch]], bv3[ch], sem)
        for ch in range(3):
            pltpu.make_async_copy(w_hbm.at[idx3[ch]], wv3[ch], sem).wait()
            pltpu.make_async_copy(b_hbm.at[idx3[ch]], bv3[ch], sem).wait()

        def comp_body(i, _):
            sl = pl.ds(i * _L, _L)
            for ch in range(3):
                t = wv3[ch][sl] * x3[ch][sl] + bv3[ch][sl]
                e = jnp.exp(t + t)
                x3[ch][sl] = 1.0 - 2.0 / (e + 1.0)
            return 0

        lax.fori_loop(0, _C // _L, comp_body, 0, unroll=2)

        for ch in range(3):
            pltpu.sync_copy(x3[ch],
                            out_hbm.at[3 * batch + ch, pl.ds(off, _C)])


@jax.jit
def _run(img2, wflat, bflat):
    mesh = plsc.VectorSubcoreMesh(core_axis_name="c", subcore_axis_name="s")
    f = pl.kernel(
        _body,
        out_type=jax.ShapeDtypeStruct((12, _S), jnp.float32),
        mesh=mesh,
        compiler_params=pltpu.CompilerParams(needs_layout_passes=False),
        scratch_types=(
            [pltpu.VMEM((_C,), jnp.float32) for _ in range(3)]
            + [pltpu.VMEM((_C,), jnp.int32) for _ in range(3)]
            + [pltpu.VMEM((_C,), jnp.float32) for _ in range(6)]
            + [pltpu.SemaphoreType.DMA]
        ),
    )
    return f(img2, wflat, bflat)


def kernel(img, weight, bias):
    img2 = img.reshape(12, _S)
    out = _run(img2, weight.reshape(-1), bias.reshape(-1))
    return out.reshape(4, 3, 512, 512)


# speedup vs baseline: 35.9710x; 35.9710x over previous
import jax
import jax.numpy as jnp
from jax import lax
from jax.experimental import pallas as pl
from jax.experimental.pallas import tpu as pltpu
from jax.experimental.pallas import tpu_sc as plsc

_TABLE_ROWS = 256 * 256 * 256
_S = 512 * 512
_P = 4 * _S
_NW = 32
_PW = _P // _NW
_C = 8192
_NCHUNK = _PW // _C
_L = 16


def _body(img_hbm, w0h, w1h, w2h, b0h, b1h, b2h, out_hbm,
          xr, xg, xb, iv, w0, w1, w2, b0, b1, b2, sem):
    nc = 2
    wid = lax.axis_index("s") * nc + lax.axis_index("c")
    batch = wid // 8
    base = (wid % 8) * _PW
    x3 = (xr, xg, xb)
    wt3 = ((w0h, w0), (w1h, w1), (w2h, w2))
    bt3 = ((b0h, b0), (b1h, b1), (b2h, b2))

    for k in range(_NCHUNK):
        off = base + k * _C

        for ch in range(3):
            pltpu.sync_copy(img_hbm.at[3 * batch + ch, pl.ds(off, _C)],
                            x3[ch])

        def idx_body(i, _):
            sl = pl.ds(i * _L, _L)
            rr = (xr[sl] + 1.0) * 127.5
            gg = (xg[sl] + 1.0) * 127.5
            bb = (xb[sl] + 1.0) * 127.5
            t = rr * 65536.0 + gg * 256.0 + bb
            iv[sl] = jnp.clip(t.astype(jnp.int32), 0, _TABLE_ROWS - 1)
            return 0

        lax.fori_loop(0, _C // _L, idx_body, 0, unroll=4)

        for hbm, dst in wt3 + bt3:
            pltpu.async_copy(hbm.at[iv], dst, sem)
        for hbm, dst in wt3 + bt3:
            pltpu.make_async_copy(hbm.at[iv], dst, sem).wait()

        def comp_body(i, _):
            sl = pl.ds(i * _L, _L)
            for ch in range(3):
                t = wt3[ch][1][sl] * x3[ch][sl] + bt3[ch][1][sl]
                e = jnp.exp(t + t)
                x3[ch][sl] = 1.0 - 2.0 / (e + 1.0)
            return 0

        lax.fori_loop(0, _C // _L, comp_body, 0, unroll=2)

        for ch in range(3):
            pltpu.sync_copy(x3[ch],
                            out_hbm.at[3 * batch + ch, pl.ds(off, _C)])


@jax.jit
def _run(img2, weight, bias):
    mesh = plsc.VectorSubcoreMesh(core_axis_name="c", subcore_axis_name="s")
    f = pl.kernel(
        _body,
        out_type=jax.ShapeDtypeStruct((12, _S), jnp.float32),
        mesh=mesh,
        compiler_params=pltpu.CompilerParams(needs_layout_passes=False),
        scratch_types=(
            [pltpu.VMEM((_C,), jnp.float32) for _ in range(3)]
            + [pltpu.VMEM((_C,), jnp.int32)]
            + [pltpu.VMEM((_C,), jnp.float32) for _ in range(6)]
            + [pltpu.SemaphoreType.DMA]
        ),
    )
    return f(img2, weight[:, 0], weight[:, 1], weight[:, 2],
             bias[:, 0], bias[:, 1], bias[:, 2])


def kernel(img, weight, bias):
    img2 = img.reshape(12, _S)
    out = _run(img2, weight, bias)
    return out.reshape(4, 3, 512, 512)


# speedup vs baseline: 38.6088x; 1.0733x over previous
import jax
import jax.numpy as jnp
from jax import lax
from jax.experimental import pallas as pl
from jax.experimental.pallas import tpu as pltpu
from jax.experimental.pallas import tpu_sc as plsc

_TABLE_ROWS = 256 * 256 * 256
_S = 512 * 512
_P = 4 * _S
_NW = 32
_PW = _P // _NW
_C = 4096
_NCHUNK = _PW // _C
_L = 16


def _body(img_hbm, w0h, w1h, w2h, b0h, b1h, b2h, out_hbm,
          xr0, xg0, xb0, xr1, xg1, xb1,
          yr0, yg0, yb0, yr1, yg1, yb1,
          iv0, iv1,
          w00, w10, w20, b00, b10, b20,
          w01, w11, w21, b01, b11, b21,
          sem_in, sem_g, sem_out):
    nc = 2
    wid = lax.axis_index("s") * nc + lax.axis_index("c")
    batch = wid // 8
    base = (wid % 8) * _PW

    xb_ = ((xr0, xg0, xb0), (xr1, xg1, xb1))
    yb_ = ((yr0, yg0, yb0), (yr1, yg1, yb1))
    ivb = (iv0, iv1)
    gb_ = (((w0h, w00), (w1h, w10), (w2h, w20),
            (b0h, b00), (b1h, b10), (b2h, b20)),
           ((w0h, w01), (w1h, w11), (w2h, w21),
            (b0h, b01), (b1h, b11), (b2h, b21)))

    def in_copies(k, p):
        off = base + k * _C
        return [pltpu.make_async_copy(
                    img_hbm.at[3 * batch + ch, pl.ds(off, _C)],
                    xb_[p][ch], sem_in)
                for ch in range(3)]

    def out_copies(k, p):
        off = base + k * _C
        return [pltpu.make_async_copy(
                    yb_[p][ch],
                    out_hbm.at[3 * batch + ch, pl.ds(off, _C)], sem_out)
                for ch in range(3)]

    def gather_copies(p):
        return [pltpu.make_async_copy(hbm.at[ivb[p]], dst, sem_g)
                for hbm, dst in gb_[p]]

    def idx_and_fire(p):
        xr, xg, xb = xb_[p]
        iv = ivb[p]

        def idx_body(i, _):
            sl = pl.ds(i * _L, _L)
            rr = (xr[sl] + 1.0) * 127.5
            gg = (xg[sl] + 1.0) * 127.5
            bb = (xb[sl] + 1.0) * 127.5
            t = rr * 65536.0 + gg * 256.0 + bb
            iv[sl] = jnp.clip(t.astype(jnp.int32), 0, _TABLE_ROWS - 1)
            return 0

        lax.fori_loop(0, _C // _L, idx_body, 0, unroll=4)
        for c in gather_copies(p):
            c.start()

    def compute(p):
        x3 = xb_[p]
        y3 = yb_[p]
        v6 = gb_[p]

        def comp_body(i, _):
            sl = pl.ds(i * _L, _L)
            for ch in range(3):
                t = v6[ch][1][sl] * x3[ch][sl] + v6[3 + ch][1][sl]
                e = jnp.exp(t + t)
                y3[ch][sl] = 1.0 - 2.0 / (e + 1.0)
            return 0

        lax.fori_loop(0, _C // _L, comp_body, 0, unroll=2)

    for c in in_copies(0, 0):
        c.start()
    for c in in_copies(0, 0):
        c.wait()
    idx_and_fire(0)
    for c in in_copies(1, 1):
        c.start()

    for k in range(_NCHUNK):
        p = k % 2
        q = (k + 1) % 2
        if k + 1 < _NCHUNK:
            for c in in_copies(k + 1, q):
                c.wait()
            idx_and_fire(q)
        for c in gather_copies(p):
            c.wait()
        if k >= 2:
            for c in out_copies(k - 2, p):
                c.wait()
        compute(p)
        for c in out_copies(k, p):
            c.start()
        if k + 2 < _NCHUNK:
            for c in in_copies(k + 2, p):
                c.start()

    for k in (_NCHUNK - 2, _NCHUNK - 1):
        for c in out_copies(k, k % 2):
            c.wait()


@jax.jit
def _run(img2, weight, bias):
    mesh = plsc.VectorSubcoreMesh(core_axis_name="c", subcore_axis_name="s")
    f = pl.kernel(
        _body,
        out_type=jax.ShapeDtypeStruct((12, _S), jnp.float32),
        mesh=mesh,
        compiler_params=pltpu.CompilerParams(needs_layout_passes=False),
        scratch_types=(
            [pltpu.VMEM((_C,), jnp.float32) for _ in range(6)]
            + [pltpu.VMEM((_C,), jnp.float32) for _ in range(6)]
            + [pltpu.VMEM((_C,), jnp.int32) for _ in range(2)]
            + [pltpu.VMEM((_C,), jnp.float32) for _ in range(12)]
            + [pltpu.SemaphoreType.DMA for _ in range(3)]
        ),
    )
    return f(img2, weight[:, 0], weight[:, 1], weight[:, 2],
             bias[:, 0], bias[:, 1], bias[:, 2])


def kernel(img, weight, bias):
    img2 = img.reshape(12, _S)
    out = _run(img2, weight, bias)
    return out.reshape(4, 3, 512, 512)


# speedup vs baseline: 41.5659x; 1.0766x over previous
import jax
import jax.numpy as jnp
from jax import lax
from jax.experimental import pallas as pl
from jax.experimental.pallas import tpu as pltpu
from jax.experimental.pallas import tpu_sc as plsc

_TABLE_ROWS = 256 * 256 * 256
_S = 512 * 512
_P = 4 * _S
_NW = 32
_PW = _P // _NW
_L = 16

_CI = 8192
_NI = _PW // _CI
_CG = 8192
_NG = _PW // _CG
_CF = 4096
_NF = _PW // _CF


def _wid():
    return lax.axis_index("s") * 2 + lax.axis_index("c")


def _idx_body(img_hbm, idx_hbm, xr, xg, xb, iv):
    wid = _wid()
    batch = wid // 8
    base = (wid % 8) * _PW
    x3 = (xr, xg, xb)

    for k in range(_NI):
        off = base + k * _CI
        for ch in range(3):
            pltpu.sync_copy(img_hbm.at[3 * batch + ch, pl.ds(off, _CI)],
                            x3[ch])

        def body(i, _):
            sl = pl.ds(i * _L, _L)
            rr = (xr[sl] + 1.0) * 127.5
            gg = (xg[sl] + 1.0) * 127.5
            bb = (xb[sl] + 1.0) * 127.5
            t = rr * 65536.0 + gg * 256.0 + bb
            iv[sl] = jnp.clip(t.astype(jnp.int32), 0, _TABLE_ROWS - 1)
            return 0

        lax.fori_loop(0, _CI // _L, body, 0, unroll=4)
        pltpu.sync_copy(iv, idx_hbm.at[pl.ds(batch * _S + off, _CI)])


def _gw_body(idx_hbm, w0h, w1h, w2h, wpx_hbm,
             iv0, iv1, v00, v10, v20, v01, v11, v21, sem_g, sem_out):
    wid = _wid()
    batch = wid // 8
    base = (wid % 8) * _PW
    ivb = (iv0, iv1)
    vb = ((v00, v10, v20), (v01, v11, v21))
    w3 = (w0h, w1h, w2h)

    def gathers(p):
        return [pltpu.make_async_copy(w3[c].at[ivb[p]], vb[p][c], sem_g)
                for c in range(3)]

    def outs(k, p):
        off = base + k * _CG
        return [pltpu.make_async_copy(
                    vb[p][c], wpx_hbm.at[pl.ds(c * _P + batch * _S + off, _CG)],
                    sem_out)
                for c in range(3)]

    pltpu.sync_copy(idx_hbm.at[pl.ds(batch * _S + base, _CG)], ivb[0])
    for c in gathers(0):
        c.start()
    for k in range(_NG):
        p = k % 2
        q = (k + 1) % 2
        if k + 1 < _NG:
            off = base + (k + 1) * _CG
            pltpu.sync_copy(idx_hbm.at[pl.ds(batch * _S + off, _CG)], ivb[q])
            if k >= 1:
                for c in outs(k - 1, q):
                    c.wait()
            for c in gathers(q):
                c.start()
        for c in gathers(p):
            c.wait()
        for c in outs(k, p):
            c.start()
    for k in (_NG - 2, _NG - 1):
        for c in outs(k, k % 2):
            c.wait()


def _fin_body(img_hbm, idx_hbm, wpx_hbm, b0h, b1h, b2h, out_hbm,
              xr0, xg0, xb0, xr1, xg1, xb1,
              yr0, yg0, yb0, yr1, yg1, yb1,
              iv0, iv1,
              w00, w10, w20, w01, w11, w21,
              b00, b10, b20, b01, b11, b21,
              sem_in, sem_g, sem_out):
    wid = _wid()
    batch = wid // 8
    base = (wid % 8) * _PW
    xb_ = ((xr0, xg0, xb0), (xr1, xg1, xb1))
    yb_ = ((yr0, yg0, yb0), (yr1, yg1, yb1))
    ivb = (iv0, iv1)
    wvb = ((w00, w10, w20), (w01, w11, w21))
    bvb = ((b00, b10, b20), (b01, b11, b21))
    b3 = (b0h, b1h, b2h)

    def in_copies(k, p):
        off = base + k * _CF
        cs = [pltpu.make_async_copy(
                  img_hbm.at[3 * batch + ch, pl.ds(off, _CF)],
                  xb_[p][ch], sem_in)
              for ch in range(3)]
        cs.append(pltpu.make_async_copy(
            idx_hbm.at[pl.ds(batch * _S + off, _CF)], ivb[p], sem_in))
        cs += [pltpu.make_async_copy(
                   wpx_hbm.at[pl.ds(c * _P + batch * _S + off, _CF)],
                   wvb[p][c], sem_in)
               for c in range(3)]
        return cs

    def gathers(p):
        return [pltpu.make_async_copy(b3[c].at[ivb[p]], bvb[p][c], sem_g)
                for c in range(3)]

    def out_copies(k, p):
        off = base + k * _CF
        return [pltpu.make_async_copy(
                    yb_[p][ch],
                    out_hbm.at[3 * batch + ch, pl.ds(off, _CF)], sem_out)
                for ch in range(3)]

    def compute(p):
        x3, y3, wv, bv = xb_[p], yb_[p], wvb[p], bvb[p]

        def body(i, _):
            sl = pl.ds(i * _L, _L)
            for ch in range(3):
                t = wv[ch][sl] * x3[ch][sl] + bv[ch][sl]
                e = jnp.exp(t + t)
                y3[ch][sl] = 1.0 - 2.0 / (e + 1.0)
            return 0

        lax.fori_loop(0, _CF // _L, body, 0, unroll=2)

    for c in in_copies(0, 0):
        c.start()
    for c in in_copies(0, 0):
        c.wait()
    for c in gathers(0):
        c.start()
    for c in in_copies(1, 1):
        c.start()

    for k in range(_NF):
        p = k % 2
        q = (k + 1) % 2
        if k + 1 < _NF:
            for c in in_copies(k + 1, q):
                c.wait()
            for c in gathers(q):
                c.start()
        for c in gathers(p):
            c.wait()
        if k >= 2:
            for c in out_copies(k - 2, p):
                c.wait()
        compute(p)
        for c in out_copies(k, p):
            c.start()
        if k + 2 < _NF:
            for c in in_copies(k + 2, p):
                c.start()

    for k in (_NF - 2, _NF - 1):
        for c in out_copies(k, k % 2):
            c.wait()


@jax.jit
def _run(img2, weight, bias):
    mesh = plsc.VectorSubcoreMesh(core_axis_name="c", subcore_axis_name="s")
    cp = pltpu.CompilerParams(needs_layout_passes=False)

    f_idx = pl.kernel(
        _idx_body,
        out_type=jax.ShapeDtypeStruct((_P,), jnp.int32),
        mesh=mesh, compiler_params=cp,
        scratch_types=(
            [pltpu.VMEM((_CI,), jnp.float32) for _ in range(3)]
            + [pltpu.VMEM((_CI,), jnp.int32)]
        ),
    )
    f_gw = pl.kernel(
        _gw_body,
        out_type=jax.ShapeDtypeStruct((3 * _P,), jnp.float32),
        mesh=mesh, compiler_params=cp,
        scratch_types=(
            [pltpu.VMEM((_CG,), jnp.int32) for _ in range(2)]
            + [pltpu.VMEM((_CG,), jnp.float32) for _ in range(6)]
            + [pltpu.SemaphoreType.DMA for _ in range(2)]
        ),
    )
    f_fin = pl.kernel(
        _fin_body,
        out_type=jax.ShapeDtypeStruct((12, _S), jnp.float32),
        mesh=mesh, compiler_params=cp,
        scratch_types=(
            [pltpu.VMEM((_CF,), jnp.float32) for _ in range(12)]
            + [pltpu.VMEM((_CF,), jnp.int32) for _ in range(2)]
            + [pltpu.VMEM((_CF,), jnp.float32) for _ in range(12)]
            + [pltpu.SemaphoreType.DMA for _ in range(3)]
        ),
    )

    w0, w1, w2 = weight[:, 0], weight[:, 1], weight[:, 2]
    b0, b1, b2 = bias[:, 0], bias[:, 1], bias[:, 2]
    idxh = f_idx(img2)
    wpx = f_gw(idxh, w0, w1, w2)
    return f_fin(img2, idxh, wpx, b0, b1, b2)


def kernel(img, weight, bias):
    img2 = img.reshape(12, _S)
    out = _run(img2, weight, bias)
    return out.reshape(4, 3, 512, 512)
